# baseline (speedup 1.0000x reference)
import functools

import jax
import jax.numpy as jnp
from jax import lax
from jax.experimental import pallas as pl
from jax.experimental.pallas import tpu as pltpu
from jax.experimental.pallas import tpu_sc as plsc

MEM_LEN = 16384
OUT_DIM = 2048
IN_DIM = 1024

NC = 2
NS = 16
NW = NC * NS
LANES = 16
JCHUNKS = OUT_DIM // LANES

SC_ROWS = 2048
TC_ROWS = MEM_LEN - SC_ROWS
ROWS_PER_W = SC_ROWS // NW
CHUNK = 16
NCHUNK = ROWS_PER_W // CHUNK
ROW_BLOCK = 1024
NBLK = TC_ROWS // ROW_BLOCK


def _fused_body(x_ref, mean_ref, std_ref, w_ref, b_ref, mem_ref,
                out_ref, e_scr, acc_ref):
    i = pl.program_id(0)

    @pl.when(i == 0)
    def _encoder():
        x = x_ref[...]
        mean = mean_ref[...]
        std = std_ref[...]
        new = (x - mean) / (std + 1e-07)
        new = jnp.where(std == 0, jnp.zeros_like(new), new)
        logits = jnp.dot(new, w_ref[...], preferred_element_type=jnp.float32)
        logits = logits + b_ref[...]
        m = jnp.max(logits, axis=-1, keepdims=True)
        shifted = logits - m
        lse = jnp.log(jnp.sum(jnp.exp(shifted), axis=-1, keepdims=True))
        e_scr[...] = shifted - lse
        acc_ref[0, 0] = jnp.inf

    d = jnp.sum(jnp.abs(mem_ref[...] - e_scr[...]), axis=1)
    blk_min = jnp.min(d)
    acc_ref[0, 0] = jnp.minimum(acc_ref[0, 0], blk_min)

    @pl.when(i == pl.num_programs(0) - 1)
    def _emit():
        out_ref[0, 0] = acc_ref[0, 0]


def _sc_scan_body(mem_hbm, e_hbm, out_hbm, e_v, buf, min_v, sem0, sem1):
    c = lax.axis_index("c")
    s = lax.axis_index("s")
    wid = s * NC + c
    base = TC_ROWS + wid * ROWS_PER_W
    pltpu.sync_copy(e_hbm, e_v)
    sems = (sem0, sem1)
    iota = lax.iota(jnp.int32, LANES)

    pltpu.make_async_copy(
        mem_hbm.at[pl.ds(base, CHUNK)], buf.at[0], sem0).start()

    def outer(g, minacc):
        for b in range(2):
            k = g * 2 + b
            nb = 1 - b
            nxt = k + 1

            @pl.when(nxt < NCHUNK)
            def _start_next():
                pltpu.make_async_copy(
                    mem_hbm.at[pl.ds(base + nxt * CHUNK, CHUNK)],
                    buf.at[nb], sems[nb]).start()

            pltpu.make_async_copy(
                mem_hbm.at[pl.ds(base + k * CHUNK, CHUNK)],
                buf.at[b], sems[b]).wait()

            def jbody(j, accs):
                e_c = e_v[pl.ds(j * LANES, LANES)]
                return tuple(
                    accs[r] + jnp.abs(buf[b, r, pl.ds(j * LANES, LANES)] - e_c)
                    for r in range(CHUNK))

            accs = lax.fori_loop(
                0, JCHUNKS, jbody,
                tuple(jnp.zeros((LANES,), jnp.float32) for _ in range(CHUNK)))
            for r in range(CHUNK):
                v = accs[r]
                for sh in (8, 4, 2, 1):
                    v = v + v[iota ^ sh]
                minacc = jnp.minimum(minacc, v)
        return minacc

    minacc = lax.fori_loop(
        0, NCHUNK // 2, outer,
        jnp.full((LANES,), jnp.inf, jnp.float32))
    min_v[...] = minacc
    pltpu.sync_copy(min_v, out_hbm.at[wid])


_sc_scan = functools.partial(
    pl.kernel,
    out_type=jax.ShapeDtypeStruct((NW, LANES), jnp.float32),
    mesh=plsc.VectorSubcoreMesh(
        core_axis_name="c", subcore_axis_name="s",
        num_cores=NC, num_subcores=NS),
    scratch_types=[
        pltpu.VMEM((OUT_DIM,), jnp.float32),
        pltpu.VMEM((2, CHUNK, OUT_DIM), jnp.float32),
        pltpu.VMEM((LANES,), jnp.float32),
        pltpu.SemaphoreType.DMA,
        pltpu.SemaphoreType.DMA,
    ],
)(_sc_scan_body)


@jax.jit
def kernel(x, mean, std, memory, W_enc, b_enc):
    mean2 = mean.reshape(1, IN_DIM)
    std2 = std.reshape(1, IN_DIM)
    b2 = b_enc.reshape(1, OUT_DIM)

    e = pl.pallas_call(
        lambda x_ref, mean_ref, std_ref, w_ref, b_ref, e_ref: _enc_only(
            x_ref, mean_ref, std_ref, w_ref, b_ref, e_ref),
        out_shape=jax.ShapeDtypeStruct((1, OUT_DIM), jnp.float32),
    )(x, mean2, std2, W_enc, b2)

    sc_partials = _sc_scan(memory, e.reshape(OUT_DIM))

    tc_min = pl.pallas_call(
        _tc_scan_only,
        grid=(NBLK,),
        in_specs=[
            pl.BlockSpec((1, OUT_DIM), lambda i: (0, 0)),
            pl.BlockSpec((ROW_BLOCK, OUT_DIM), lambda i: (i, 0)),
        ],
        out_specs=pl.BlockSpec(memory_space=pltpu.SMEM),
        out_shape=jax.ShapeDtypeStruct((1, 1), jnp.float32),
        scratch_shapes=[pltpu.SMEM((1, 1), jnp.float32)],
    )(e, memory)

    return jnp.minimum(tc_min[0, 0], jnp.min(sc_partials))


def _enc_only(x_ref, mean_ref, std_ref, w_ref, b_ref, e_ref):
    x = x_ref[...]
    mean = mean_ref[...]
    std = std_ref[...]
    new = (x - mean) / (std + 1e-07)
    new = jnp.where(std == 0, jnp.zeros_like(new), new)
    logits = jnp.dot(new, w_ref[...], preferred_element_type=jnp.float32)
    logits = logits + b_ref[...]
    m = jnp.max(logits, axis=-1, keepdims=True)
    shifted = logits - m
    lse = jnp.log(jnp.sum(jnp.exp(shifted), axis=-1, keepdims=True))
    e_ref[...] = shifted - lse


def _tc_scan_only(e_ref, mem_ref, out_ref, acc_ref):
    i = pl.program_id(0)

    @pl.when(i == 0)
    def _init():
        acc_ref[0, 0] = jnp.inf

    d = jnp.sum(jnp.abs(mem_ref[...] - e_ref[...]), axis=1)
    acc_ref[0, 0] = jnp.minimum(acc_ref[0, 0], jnp.min(d))

    @pl.when(i == pl.num_programs(0) - 1)
    def _emit():
        out_ref[0, 0] = acc_ref[0, 0]


# speedup vs baseline: 1.4456x; 1.4456x over previous
import jax
import jax.numpy as jnp
from jax.experimental import pallas as pl
from jax.experimental.pallas import tpu as pltpu

MEM_LEN = 16384
OUT_DIM = 2048
IN_DIM = 1024
ROW_BLOCK = 1024
NBLK = MEM_LEN // ROW_BLOCK


def _fused_body(x_ref, mean_ref, std_ref, w_ref, b_ref, memt_ref, memb_ref,
                out_ref, e_scr, acc_ref):
    i = pl.program_id(0)

    @pl.when(i == 0)
    def _encoder():
        x = x_ref[...]
        mean = mean_ref[...]
        std = std_ref[...]
        new = (x - mean) / (std + 1e-07)
        new = jnp.where(std == 0, jnp.zeros_like(new), new)
        logits = jnp.dot(new, w_ref[...], preferred_element_type=jnp.float32)
        logits = logits + b_ref[...]
        m = jnp.max(logits, axis=-1, keepdims=True)
        shifted = logits - m
        lse = jnp.log(jnp.sum(jnp.exp(shifted), axis=-1, keepdims=True))
        e_scr[...] = shifted - lse
        acc_ref[0, 0] = jnp.inf

    d = jnp.minimum(
        jnp.sum(jnp.abs(memt_ref[...] - e_scr[...]), axis=1),
        jnp.sum(jnp.abs(memb_ref[...] - e_scr[...]), axis=1))
    blk_min = jnp.min(d)
    acc_ref[0, 0] = jnp.minimum(acc_ref[0, 0], blk_min)

    @pl.when(i == pl.num_programs(0) - 1)
    def _emit():
        out_ref[0, 0] = acc_ref[0, 0]


@jax.jit
def kernel(x, mean, std, memory, W_enc, b_enc):
    mean2 = mean.reshape(1, IN_DIM)
    std2 = std.reshape(1, IN_DIM)
    b2 = b_enc.reshape(1, OUT_DIM)

    zero = lambda i: (0, 0)
    mem_idx = lambda i: (i, 0)

    out = pl.pallas_call(
        _fused_body,
        grid=(NBLK // 2,),
        in_specs=[
            pl.BlockSpec((1, IN_DIM), zero),
            pl.BlockSpec((1, IN_DIM), zero),
            pl.BlockSpec((1, IN_DIM), zero),
            pl.BlockSpec((IN_DIM, OUT_DIM), zero),
            pl.BlockSpec((1, OUT_DIM), zero),
            pl.BlockSpec((ROW_BLOCK, OUT_DIM), mem_idx),
            pl.BlockSpec((ROW_BLOCK, OUT_DIM), lambda i: (i + NBLK // 2, 0)),
        ],
        out_specs=pl.BlockSpec(memory_space=pltpu.SMEM),
        out_shape=jax.ShapeDtypeStruct((1, 1), jnp.float32),
        scratch_shapes=[
            pltpu.VMEM((1, OUT_DIM), jnp.float32),
            pltpu.SMEM((1, 1), jnp.float32),
        ],
    )(x, mean2, std2, W_enc, b2, memory, memory)
    return out[0, 0]


# speedup vs baseline: 1.4874x; 1.0289x over previous
import jax
import jax.numpy as jnp
from jax.experimental import pallas as pl
from jax.experimental.pallas import tpu as pltpu

MEM_LEN = 16384
OUT_DIM = 2048
IN_DIM = 1024
ROW_BLOCK = 1024
NBLK = MEM_LEN // ROW_BLOCK


def _fused_body(x_ref, mean_ref, std_ref, w_ref, b_ref, mem_ref,
                out_ref, e_scr, acc_ref):
    i = pl.program_id(0)

    @pl.when(i == 0)
    def _encoder():
        x = x_ref[...]
        mean = mean_ref[...]
        std = std_ref[...]
        new = (x - mean) / (std + 1e-07)
        new = jnp.where(std == 0, jnp.zeros_like(new), new)
        logits = jnp.dot(new, w_ref[...], preferred_element_type=jnp.float32)
        logits = logits + b_ref[...]
        m = jnp.max(logits, axis=-1, keepdims=True)
        shifted = logits - m
        lse = jnp.log(jnp.sum(jnp.exp(shifted), axis=-1, keepdims=True))
        e_scr[...] = shifted - lse
        acc_ref[0, 0] = jnp.inf

    d = jnp.sum(jnp.abs(mem_ref[...] - e_scr[...]), axis=1)
    blk_min = jnp.min(d)
    acc_ref[0, 0] = jnp.minimum(acc_ref[0, 0], blk_min)

    @pl.when(i == pl.num_programs(0) - 1)
    def _emit():
        out_ref[0, 0] = acc_ref[0, 0]


@jax.jit
def kernel(x, mean, std, memory, W_enc, b_enc):
    mean2 = mean.reshape(1, IN_DIM)
    std2 = std.reshape(1, IN_DIM)
    b2 = b_enc.reshape(1, OUT_DIM)

    zero = lambda i: (0, 0)
    mem_idx = lambda i: (i, 0)

    out = pl.pallas_call(
        _fused_body,
        grid=(NBLK,),
        in_specs=[
            pl.BlockSpec((1, IN_DIM), zero),
            pl.BlockSpec((1, IN_DIM), zero),
            pl.BlockSpec((1, IN_DIM), zero),
            pl.BlockSpec((IN_DIM, OUT_DIM), zero),
            pl.BlockSpec((1, OUT_DIM), zero),
            pl.BlockSpec((ROW_BLOCK, OUT_DIM), mem_idx),
        ],
        out_specs=pl.BlockSpec(memory_space=pltpu.SMEM),
        out_shape=jax.ShapeDtypeStruct((1, 1), jnp.float32),
        scratch_shapes=[
            pltpu.VMEM((1, OUT_DIM), jnp.float32),
            pltpu.SMEM((1, 1), jnp.float32),
        ],
    )(x, mean2, std2, W_enc, b2, memory)
    return out[0, 0]
